# Initial kernel scaffold; baseline (speedup 1.0000x reference)
import jax
import jax.numpy as jnp
from jax.experimental import pallas as pl


def kernel(feature, edge_index, W1, b1, Wm, bm, Wl, bl):
    raise NotImplementedError("write your pallas kernel here")


# speedup vs baseline: 6.3227x; 6.3227x over previous
import functools

import jax
import jax.numpy as jnp
from jax import lax
from jax.experimental import pallas as pl
from jax.experimental.pallas import tpu as pltpu
from jax.experimental.pallas import tpu_sc as plsc

N = 10000
F = 256
H = 256
L = 128
E = 160000

R = 10240
EP = 163840
EDGES_PER_TILE = EP // 16
BLOCKS_PER_TILE = EDGES_PER_TILE // 128
DEG_CHUNK = EP // 32
ROWS_PER_TILE = R // 16
BM = 1024
NB = R // BM


def _mesh():
    return plsc.VectorSubcoreMesh(core_axis_name="c", subcore_axis_name="s")


def _deg_body(dsts_hbm, out_hbm, dst_v, hist_v, tmp_v, acc_v, shared):
    c = lax.axis_index("c")
    s = lax.axis_index("s")
    w = c * 16 + s
    z16 = jnp.zeros((16,), jnp.float32)

    def zero_hist(i, _):
        hist_v[pl.ds(i * 16, 16)] = z16
        return 0
    lax.fori_loop(0, R // 16, zero_hist, 0)

    pltpu.sync_copy(dsts_hbm.at[pl.ds(w * DEG_CHUNK, DEG_CHUNK)], dst_v)
    ones = jnp.ones((16,), jnp.float32)

    def hist_step(i, _):
        idx = dst_v[pl.ds(i * 16, 16)]
        plsc.addupdate_scatter(hist_v, [idx], ones)
        return 0
    lax.fori_loop(0, DEG_CHUNK // 16, hist_step, 0)

    pltpu.sync_copy(hist_v, shared.at[s])
    plsc.subcore_barrier()

    def zero_acc(i, _):
        acc_v[pl.ds(i * 16, 16)] = z16
        return 0
    lax.fori_loop(0, ROWS_PER_TILE // 16, zero_acc, 0)
    for t in range(16):
        pltpu.sync_copy(shared.at[t, pl.ds(s * ROWS_PER_TILE, ROWS_PER_TILE)], tmp_v)

        def add_step(j, _):
            acc_v[pl.ds(j * 16, 16)] = acc_v[pl.ds(j * 16, 16)] + tmp_v[pl.ds(j * 16, 16)]
            return 0
        lax.fori_loop(0, ROWS_PER_TILE // 16, add_step, 0)
    pltpu.sync_copy(acc_v, out_hbm.at[c, pl.ds(s * ROWS_PER_TILE, ROWS_PER_TILE)])


def _sc_degree(dsts):
    f = functools.partial(
        pl.kernel,
        out_type=jax.ShapeDtypeStruct((2, R), jnp.float32),
        mesh=_mesh(),
        scratch_types=[
            pltpu.VMEM((DEG_CHUNK,), jnp.int32),
            pltpu.VMEM((R,), jnp.float32),
            pltpu.VMEM((ROWS_PER_TILE,), jnp.float32),
            pltpu.VMEM((ROWS_PER_TILE,), jnp.float32),
            pltpu.VMEM_SHARED((16, R), jnp.float32),
        ],
        compiler_params=pltpu.CompilerParams(needs_layout_passes=False),
    )(_deg_body)
    return f(dsts)


def _prop_body(table_hbm, srcs_hbm, dsts_hbm, zeros_hbm, out_hbm,
               idx_v, dst_v, rows_v, accum, gsem, ssem):
    c = lax.axis_index("c")
    s = lax.axis_index("s")
    row0 = s * ROWS_PER_TILE
    pltpu.sync_copy(zeros_hbm.at[pl.ds(row0, ROWS_PER_TILE)],
                    accum.at[pl.ds(row0, ROWS_PER_TILE)])
    plsc.subcore_barrier()

    def blk(k, _):
        pltpu.sync_copy(srcs_hbm.at[c, s, k], idx_v)
        pltpu.sync_copy(dsts_hbm.at[s, k], dst_v)
        pltpu.async_copy(table_hbm.at[idx_v], rows_v, gsem).wait()
        pltpu.async_copy(rows_v, accum.at[dst_v], ssem, add=True).wait()
        return 0
    lax.fori_loop(0, BLOCKS_PER_TILE, blk, 0)

    plsc.subcore_barrier()
    pltpu.sync_copy(accum.at[pl.ds(row0, ROWS_PER_TILE)],
                    out_hbm.at[c, pl.ds(row0, ROWS_PER_TILE)])


def _sc_propagate(table_flat, srcs, dsts, zeros_rows):
    f = functools.partial(
        pl.kernel,
        out_type=jax.ShapeDtypeStruct((2, R, L), jnp.float32),
        mesh=_mesh(),
        scratch_types=[
            pltpu.VMEM((128,), jnp.int32),
            pltpu.VMEM((128,), jnp.int32),
            pltpu.VMEM((128, L), jnp.float32),
            pltpu.VMEM_SHARED((R, L), jnp.float32),
            pltpu.SemaphoreType.DMA,
            pltpu.SemaphoreType.DMA,
        ],
        compiler_params=pltpu.CompilerParams(needs_layout_passes=False),
    )(_prop_body)
    return f(table_flat, srcs, dsts, zeros_rows)


def _mm_body(x_ref, w_ref, o_ref):
    o_ref[...] = jnp.dot(x_ref[...], w_ref[...], preferred_element_type=jnp.float32)


def _tc_matmul(x, w):
    return pl.pallas_call(
        _mm_body,
        grid=(NB,),
        in_specs=[
            pl.BlockSpec((BM, F), lambda i: (i, 0)),
            pl.BlockSpec((F, H), lambda i: (0, 0)),
        ],
        out_specs=pl.BlockSpec((BM, H), lambda i: (i, 0)),
        out_shape=jax.ShapeDtypeStruct((R, H), jnp.float32),
    )(x, w)


def _scale_body(u_ref, da_ref, db_ref, us_ref, dinv_ref):
    i = pl.program_id(0)
    rows = lax.broadcasted_iota(jnp.int32, (BM, 1), 0) + i * BM
    mask = (rows < N).astype(jnp.float32)
    dinv = lax.rsqrt(da_ref[...] + db_ref[...] + 1.0)
    dinv_ref[...] = dinv
    us_ref[0] = mask * dinv * u_ref[...]


def _tc_scale(u, deg_a, deg_b):
    return pl.pallas_call(
        _scale_body,
        grid=(NB, 2),
        in_specs=[
            pl.BlockSpec((BM, L), lambda i, c: (i, c)),
            pl.BlockSpec((BM, 1), lambda i, c: (i, 0)),
            pl.BlockSpec((BM, 1), lambda i, c: (i, 0)),
        ],
        out_specs=[
            pl.BlockSpec((1, BM, L), lambda i, c: (c, i, 0)),
            pl.BlockSpec((BM, 1), lambda i, c: (i, 0)),
        ],
        out_shape=[
            jax.ShapeDtypeStruct((2, R, L), jnp.float32),
            jax.ShapeDtypeStruct((R, 1), jnp.float32),
        ],
    )(u, deg_a, deg_b)


def _layer1_body(s1_ref, us_ref, dinv_ref, b1_ref, hs_ref):
    i = pl.program_id(0)
    rows = lax.broadcasted_iota(jnp.int32, (BM, 1), 0) + i * BM
    mask = (rows < N).astype(jnp.float32)
    dinv = dinv_ref[...]
    t = dinv * (s1_ref[0] + us_ref[0]) + b1_ref[0]
    hs_ref[0] = mask * dinv * jnp.tanh(t)


def _tc_layer1(s1, us3, dinv, b1_2):
    return pl.pallas_call(
        _layer1_body,
        grid=(NB, 2),
        in_specs=[
            pl.BlockSpec((1, BM, L), lambda i, c: (c, i, 0)),
            pl.BlockSpec((1, BM, L), lambda i, c: (c, i, 0)),
            pl.BlockSpec((BM, 1), lambda i, c: (i, 0)),
            pl.BlockSpec((1, 1, L), lambda i, c: (c, 0, 0)),
        ],
        out_specs=pl.BlockSpec((1, BM, L), lambda i, c: (c, i, 0)),
        out_shape=jax.ShapeDtypeStruct((2, R, L), jnp.float32),
    )(s1, us3, dinv, b1_2)


def _head_body(s2_ref, hs_ref, dinv_ref, wm_ref, wl_ref, bm_ref, bl_ref,
               noise_ref, z_ref, mean_ref, lv_ref):
    dinv = dinv_ref[...]
    p0 = dinv * (s2_ref[0] + hs_ref[0])
    p1 = dinv * (s2_ref[1] + hs_ref[1])
    mean = (jnp.dot(p0, wm_ref[0], preferred_element_type=jnp.float32)
            + jnp.dot(p1, wm_ref[1], preferred_element_type=jnp.float32)
            + bm_ref[...])
    lv = (jnp.dot(p0, wl_ref[0], preferred_element_type=jnp.float32)
          + jnp.dot(p1, wl_ref[1], preferred_element_type=jnp.float32)
          + bl_ref[...])
    mean_ref[...] = mean
    lv_ref[...] = lv
    z_ref[...] = noise_ref[...] * jnp.exp(0.5 * lv) + mean


def _tc_head(s2, hs3, dinv, wm3, wl3, bm_2, bl_2, noise):
    spec_rl = pl.BlockSpec((2, BM, L), lambda i: (0, i, 0))
    return pl.pallas_call(
        _head_body,
        grid=(NB,),
        in_specs=[
            spec_rl,
            spec_rl,
            pl.BlockSpec((BM, 1), lambda i: (i, 0)),
            pl.BlockSpec((2, L, L), lambda i: (0, 0, 0)),
            pl.BlockSpec((2, L, L), lambda i: (0, 0, 0)),
            pl.BlockSpec((1, L), lambda i: (0, 0)),
            pl.BlockSpec((1, L), lambda i: (0, 0)),
            pl.BlockSpec((BM, L), lambda i: (i, 0)),
        ],
        out_specs=[
            pl.BlockSpec((BM, L), lambda i: (i, 0)),
            pl.BlockSpec((BM, L), lambda i: (i, 0)),
            pl.BlockSpec((BM, L), lambda i: (i, 0)),
        ],
        out_shape=[
            jax.ShapeDtypeStruct((R, L), jnp.float32),
            jax.ShapeDtypeStruct((R, L), jnp.float32),
            jax.ShapeDtypeStruct((R, L), jnp.float32),
        ],
    )(s2, hs3, dinv, wm3, wl3, bm_2, bl_2, noise)


@jax.jit
def _run(feature, edge_index, W1, b1, Wm, bm, Wl, bl):
    src = edge_index[0]
    dst = edge_index[1]
    pad = jnp.full((EP - E,), N, dtype=jnp.int32)
    src_p = jnp.concatenate([src, pad])
    dst_p = jnp.concatenate([dst, pad])
    srcs = jnp.stack([src_p, src_p + R]).reshape(2, 16, BLOCKS_PER_TILE, 128)
    dsts_blocked = dst_p.reshape(16, BLOCKS_PER_TILE, 128)

    xp = jnp.zeros((R, F), jnp.float32).at[:N].set(feature)
    zeros_rows = jnp.zeros((R, L), jnp.float32)
    noise = jax.random.normal(jax.random.key(42), (N, L), dtype=jnp.float32)
    noise_p = jnp.zeros((R, L), jnp.float32).at[:N].set(noise)

    deg2 = _sc_degree(dst_p)
    deg_a = deg2[0].reshape(R, 1)
    deg_b = deg2[1].reshape(R, 1)

    u = _tc_matmul(xp, W1)
    us3, dinv = _tc_scale(u, deg_a, deg_b)

    s1 = _sc_propagate(us3.reshape(2 * R, L), srcs, dsts_blocked, zeros_rows)
    hs3 = _tc_layer1(s1, us3, dinv, b1.reshape(2, 1, L))
    s2 = _sc_propagate(hs3.reshape(2 * R, L), srcs, dsts_blocked, zeros_rows)

    z, mean, lv = _tc_head(s2, hs3, dinv, Wm.reshape(2, L, L), Wl.reshape(2, L, L),
                           bm.reshape(1, L), bl.reshape(1, L), noise_p)
    return z[:N], mean[:N], lv[:N]


def kernel(feature, edge_index, W1, b1, Wm, bm, Wl, bl):
    return _run(feature, edge_index, W1, b1, Wm, bm, Wl, bl)


# speedup vs baseline: 8.4452x; 1.3357x over previous
import functools

import jax
import jax.numpy as jnp
from jax import lax
from jax.experimental import pallas as pl
from jax.experimental.pallas import tpu as pltpu
from jax.experimental.pallas import tpu_sc as plsc

N = 10000
F = 256
H = 256
L = 128
E = 160000

R = 10240
EP = 163840
EDGES_PER_TILE = EP // 16
BSZ = 128
BLOCKS_PER_TILE = EDGES_PER_TILE // BSZ
CQ = 64
DEG_CHUNK = EP // 32
ROWS_PER_TILE = R // 16
BM = 1024
NB = R // BM


def _mesh():
    return plsc.VectorSubcoreMesh(core_axis_name="c", subcore_axis_name="s")


def _deg_body(dsts_hbm, out_hbm, dst_v, hist_v, tmp_v, acc_v, shared):
    c = lax.axis_index("c")
    s = lax.axis_index("s")
    w = c * 16 + s
    z16 = jnp.zeros((16,), jnp.float32)

    def zero_hist(i, _):
        hist_v[pl.ds(i * 16, 16)] = z16
        return 0
    lax.fori_loop(0, R // 16, zero_hist, 0)

    pltpu.sync_copy(dsts_hbm.at[pl.ds(w * DEG_CHUNK, DEG_CHUNK)], dst_v)
    ones = jnp.ones((16,), jnp.float32)

    def hist_step(i, _):
        idx = dst_v[pl.ds(i * 16, 16)]
        plsc.addupdate_scatter(hist_v, [idx], ones)
        return 0
    lax.fori_loop(0, DEG_CHUNK // 16, hist_step, 0)

    pltpu.sync_copy(hist_v, shared.at[s])
    plsc.subcore_barrier()

    def zero_acc(i, _):
        acc_v[pl.ds(i * 16, 16)] = z16
        return 0
    lax.fori_loop(0, ROWS_PER_TILE // 16, zero_acc, 0)
    for t in range(16):
        pltpu.sync_copy(shared.at[t, pl.ds(s * ROWS_PER_TILE, ROWS_PER_TILE)], tmp_v)

        def add_step(j, _):
            acc_v[pl.ds(j * 16, 16)] = acc_v[pl.ds(j * 16, 16)] + tmp_v[pl.ds(j * 16, 16)]
            return 0
        lax.fori_loop(0, ROWS_PER_TILE // 16, add_step, 0)
    pltpu.sync_copy(acc_v, out_hbm.at[c, pl.ds(s * ROWS_PER_TILE, ROWS_PER_TILE)])


def _sc_degree(dsts):
    f = functools.partial(
        pl.kernel,
        out_type=jax.ShapeDtypeStruct((2, R), jnp.float32),
        mesh=_mesh(),
        scratch_types=[
            pltpu.VMEM((DEG_CHUNK,), jnp.int32),
            pltpu.VMEM((R,), jnp.float32),
            pltpu.VMEM((ROWS_PER_TILE,), jnp.float32),
            pltpu.VMEM((ROWS_PER_TILE,), jnp.float32),
            pltpu.VMEM_SHARED((16, R), jnp.float32),
        ],
        compiler_params=pltpu.CompilerParams(needs_layout_passes=False),
    )(_deg_body)
    return f(dsts)


NBUF = 4


def _prop_body(table_hbm, srcs_hbm, dsts_hbm, zeros_hbm, out_hbm,
               src_all, dst_all, rows, accum, *sems):
    gsems = sems[:NBUF]
    ssems = sems[NBUF:]
    c = lax.axis_index("c")
    s = lax.axis_index("s")
    row0 = s * ROWS_PER_TILE
    pltpu.sync_copy(zeros_hbm.at[pl.ds(row0, ROWS_PER_TILE)],
                    accum.at[pl.ds(row0, ROWS_PER_TILE)])
    pltpu.sync_copy(srcs_hbm.at[c, s], src_all)
    pltpu.sync_copy(dsts_hbm.at[s], dst_all)
    plsc.subcore_barrier()

    def gather(k, b):
        return pltpu.async_copy(table_hbm.at[src_all.at[k]], rows.at[b],
                                gsems[b])

    def scatter(k, b):
        return pltpu.async_copy(rows.at[b], accum.at[dst_all.at[k]],
                                ssems[b], add=True)

    for b in range(NBUF):
        gather(b, b)

    def group(k0, _):
        for b in range(NBUF):
            k = k0 * NBUF + b
            pltpu.make_async_copy(table_hbm.at[src_all.at[k]], rows.at[b],
                                  gsems[b]).wait()
            scatter(k, b)
            pltpu.make_async_copy(rows.at[b], accum.at[dst_all.at[k]],
                                  ssems[b]).wait()

            @pl.when(k0 < BLOCKS_PER_TILE // NBUF - 1)
            def _():
                gather(k + NBUF, b)
        return 0
    lax.fori_loop(0, BLOCKS_PER_TILE // NBUF, group, 0)

    plsc.subcore_barrier()
    pltpu.sync_copy(accum.at[pl.ds(row0, ROWS_PER_TILE)],
                    out_hbm.at[c, pl.ds(row0, ROWS_PER_TILE)])


def _sc_propagate(table_flat, srcs, dsts, zeros_rows):
    f = functools.partial(
        pl.kernel,
        out_type=jax.ShapeDtypeStruct((2, R, CQ), jnp.float32),
        mesh=_mesh(),
        scratch_types=[
            pltpu.VMEM((BLOCKS_PER_TILE, BSZ), jnp.int32),
            pltpu.VMEM((BLOCKS_PER_TILE, BSZ), jnp.int32),
            pltpu.VMEM((NBUF, BSZ, CQ), jnp.float32),
            pltpu.VMEM_SHARED((R, CQ), jnp.float32),
        ] + [pltpu.SemaphoreType.DMA] * (2 * NBUF),
        compiler_params=pltpu.CompilerParams(needs_layout_passes=False,
                                             use_tc_tiling_on_sc=False),
    )(_prop_body)
    return f(table_flat, srcs, dsts, zeros_rows)


def _mm_body(x_ref, w_ref, o_ref):
    o_ref[...] = jnp.dot(x_ref[...], w_ref[...], preferred_element_type=jnp.float32)


def _tc_matmul(x, w):
    return pl.pallas_call(
        _mm_body,
        grid=(NB,),
        in_specs=[
            pl.BlockSpec((BM, F), lambda i: (i, 0)),
            pl.BlockSpec((F, H), lambda i: (0, 0)),
        ],
        out_specs=pl.BlockSpec((BM, H), lambda i: (i, 0)),
        out_shape=jax.ShapeDtypeStruct((R, H), jnp.float32),
    )(x, w)


def _scale_body(u_ref, da_ref, db_ref, us_ref, dinv_ref):
    i = pl.program_id(0)
    rows = lax.broadcasted_iota(jnp.int32, (BM, 1), 0) + i * BM
    mask = (rows < N).astype(jnp.float32)
    dinv = lax.rsqrt(da_ref[...] + db_ref[...] + 1.0)
    dinv_ref[...] = dinv
    md = mask * dinv
    for q in range(4):
        us_ref[q] = md * u_ref[:, q * CQ:(q + 1) * CQ]


def _tc_scale(u, deg_a, deg_b):
    return pl.pallas_call(
        _scale_body,
        grid=(NB,),
        in_specs=[
            pl.BlockSpec((BM, H), lambda i: (i, 0)),
            pl.BlockSpec((BM, 1), lambda i: (i, 0)),
            pl.BlockSpec((BM, 1), lambda i: (i, 0)),
        ],
        out_specs=[
            pl.BlockSpec((4, BM, CQ), lambda i: (0, i, 0)),
            pl.BlockSpec((BM, 1), lambda i: (i, 0)),
        ],
        out_shape=[
            jax.ShapeDtypeStruct((4, R, CQ), jnp.float32),
            jax.ShapeDtypeStruct((R, 1), jnp.float32),
        ],
    )(u, deg_a, deg_b)


def _layer1_body(s1_ref, us_ref, dinv_ref, b1_ref, hs_ref):
    i = pl.program_id(0)
    rows = lax.broadcasted_iota(jnp.int32, (BM, 1), 0) + i * BM
    mask = (rows < N).astype(jnp.float32)
    dinv = dinv_ref[...]
    md = mask * dinv
    for q in range(4):
        t = dinv * (s1_ref[q] + us_ref[q]) + b1_ref[q]
        hs_ref[q] = md * jnp.tanh(t)


def _tc_layer1(s1, us3, dinv, b1_2):
    spec_q = pl.BlockSpec((4, BM, CQ), lambda i: (0, i, 0))
    return pl.pallas_call(
        _layer1_body,
        grid=(NB,),
        in_specs=[
            spec_q,
            spec_q,
            pl.BlockSpec((BM, 1), lambda i: (i, 0)),
            pl.BlockSpec((4, 1, CQ), lambda i: (0, 0, 0)),
        ],
        out_specs=spec_q,
        out_shape=jax.ShapeDtypeStruct((4, R, CQ), jnp.float32),
    )(s1, us3, dinv, b1_2)


def _head_body(s2_ref, hs_ref, dinv_ref, wm_ref, wl_ref, bm_ref, bl_ref,
               noise_ref, z_ref, mean_ref, lv_ref):
    dinv = dinv_ref[...]
    mean = bm_ref[...]
    lv = bl_ref[...]
    for q in range(4):
        pq = dinv * (s2_ref[q] + hs_ref[q])
        mean = mean + jnp.dot(pq, wm_ref[q], preferred_element_type=jnp.float32)
        lv = lv + jnp.dot(pq, wl_ref[q], preferred_element_type=jnp.float32)
    mean_ref[...] = mean
    lv_ref[...] = lv
    z_ref[...] = noise_ref[...] * jnp.exp(0.5 * lv) + mean


def _tc_head(s2, hs3, dinv, wm3, wl3, bm_2, bl_2, noise):
    spec_rl = pl.BlockSpec((4, BM, CQ), lambda i: (0, i, 0))
    return pl.pallas_call(
        _head_body,
        grid=(NB,),
        in_specs=[
            spec_rl,
            spec_rl,
            pl.BlockSpec((BM, 1), lambda i: (i, 0)),
            pl.BlockSpec((4, CQ, L), lambda i: (0, 0, 0)),
            pl.BlockSpec((4, CQ, L), lambda i: (0, 0, 0)),
            pl.BlockSpec((1, L), lambda i: (0, 0)),
            pl.BlockSpec((1, L), lambda i: (0, 0)),
            pl.BlockSpec((BM, L), lambda i: (i, 0)),
        ],
        out_specs=[
            pl.BlockSpec((BM, L), lambda i: (i, 0)),
            pl.BlockSpec((BM, L), lambda i: (i, 0)),
            pl.BlockSpec((BM, L), lambda i: (i, 0)),
        ],
        out_shape=[
            jax.ShapeDtypeStruct((R, L), jnp.float32),
            jax.ShapeDtypeStruct((R, L), jnp.float32),
            jax.ShapeDtypeStruct((R, L), jnp.float32),
        ],
    )(s2, hs3, dinv, wm3, wl3, bm_2, bl_2, noise)


@jax.jit
def _run(feature, edge_index, W1, b1, Wm, bm, Wl, bl):
    src = edge_index[0]
    dst = edge_index[1]
    pad = jnp.full((EP - E,), N, dtype=jnp.int32)
    src_p = jnp.concatenate([src, pad])
    dst_p = jnp.concatenate([dst, pad])
    srcs4 = (src_p[None, :]
             + (jnp.arange(4, dtype=jnp.int32) * R)[:, None]
             ).reshape(4, 16, BLOCKS_PER_TILE, BSZ)
    dsts_blocked = dst_p.reshape(16, BLOCKS_PER_TILE, BSZ)

    xp = jnp.zeros((R, F), jnp.float32).at[:N].set(feature)
    zeros_rows = jnp.zeros((R, CQ), jnp.float32)
    noise = jax.random.normal(jax.random.key(42), (N, L), dtype=jnp.float32)
    noise_p = jnp.zeros((R, L), jnp.float32).at[:N].set(noise)

    deg2 = _sc_degree(dst_p)
    deg_a = deg2[0].reshape(R, 1)
    deg_b = deg2[1].reshape(R, 1)

    u = _tc_matmul(xp, W1)
    us4, dinv = _tc_scale(u, deg_a, deg_b)

    ut = us4.reshape(4 * R, CQ)
    s1 = jnp.concatenate([
        _sc_propagate(ut, srcs4[:2], dsts_blocked, zeros_rows),
        _sc_propagate(ut, srcs4[2:], dsts_blocked, zeros_rows),
    ], axis=0)
    hs4 = _tc_layer1(s1, us4, dinv, b1.reshape(4, 1, CQ))
    ht = hs4.reshape(4 * R, CQ)
    s2 = jnp.concatenate([
        _sc_propagate(ht, srcs4[:2], dsts_blocked, zeros_rows),
        _sc_propagate(ht, srcs4[2:], dsts_blocked, zeros_rows),
    ], axis=0)

    z, mean, lv = _tc_head(s2, hs4, dinv, Wm.reshape(4, CQ, L), Wl.reshape(4, CQ, L),
                           bm.reshape(1, L), bl.reshape(1, L), noise_p)
    return z[:N], mean[:N], lv[:N]


def kernel(feature, edge_index, W1, b1, Wm, bm, Wl, bl):
    return _run(feature, edge_index, W1, b1, Wm, bm, Wl, bl)


# speedup vs baseline: 8.4559x; 1.0013x over previous
import functools

import jax
import jax.numpy as jnp
from jax import lax
from jax.experimental import pallas as pl
from jax.experimental.pallas import tpu as pltpu
from jax.experimental.pallas import tpu_sc as plsc

N = 10000
F = 256
H = 256
L = 128
E = 160000

R = 10240
EP = 163840
EDGES_PER_TILE = EP // 16
BSZ = 128
BLOCKS_PER_TILE = EDGES_PER_TILE // BSZ
CQ = 64
DEG_CHUNK = EP // 32
ROWS_PER_TILE = R // 16
BM = 1024
NB = R // BM


def _mesh():
    return plsc.VectorSubcoreMesh(core_axis_name="c", subcore_axis_name="s")


def _deg_body(dsts_hbm, out_hbm, dst_v, hist_v, tmp_v, acc_v, shared):
    c = lax.axis_index("c")
    s = lax.axis_index("s")
    w = c * 16 + s
    z16 = jnp.zeros((16,), jnp.float32)

    def zero_hist(i, _):
        hist_v[pl.ds(i * 16, 16)] = z16
        return 0
    lax.fori_loop(0, R // 16, zero_hist, 0)

    pltpu.sync_copy(dsts_hbm.at[pl.ds(w * DEG_CHUNK, DEG_CHUNK)], dst_v)
    ones = jnp.ones((16,), jnp.float32)

    def hist_step(i, _):
        idx = dst_v[pl.ds(i * 16, 16)]
        plsc.addupdate_scatter(hist_v, [idx], ones)
        return 0
    lax.fori_loop(0, DEG_CHUNK // 16, hist_step, 0)

    pltpu.sync_copy(hist_v, shared.at[s])
    plsc.subcore_barrier()

    def zero_acc(i, _):
        acc_v[pl.ds(i * 16, 16)] = z16
        return 0
    lax.fori_loop(0, ROWS_PER_TILE // 16, zero_acc, 0)
    for t in range(16):
        pltpu.sync_copy(shared.at[t, pl.ds(s * ROWS_PER_TILE, ROWS_PER_TILE)], tmp_v)

        def add_step(j, _):
            acc_v[pl.ds(j * 16, 16)] = acc_v[pl.ds(j * 16, 16)] + tmp_v[pl.ds(j * 16, 16)]
            return 0
        lax.fori_loop(0, ROWS_PER_TILE // 16, add_step, 0)
    pltpu.sync_copy(acc_v, out_hbm.at[c, pl.ds(s * ROWS_PER_TILE, ROWS_PER_TILE)])


def _sc_degree(dsts):
    f = functools.partial(
        pl.kernel,
        out_type=jax.ShapeDtypeStruct((2, R), jnp.float32),
        mesh=_mesh(),
        scratch_types=[
            pltpu.VMEM((DEG_CHUNK,), jnp.int32),
            pltpu.VMEM((R,), jnp.float32),
            pltpu.VMEM((ROWS_PER_TILE,), jnp.float32),
            pltpu.VMEM((ROWS_PER_TILE,), jnp.float32),
            pltpu.VMEM_SHARED((16, R), jnp.float32),
        ],
        compiler_params=pltpu.CompilerParams(needs_layout_passes=False),
    )(_deg_body)
    return f(dsts)


NBUF = 5


def _prop_body(table_hbm, srcs_hbm, dsts_hbm, zeros_hbm, out_hbm,
               src_all, dst_all, rows, accum, *sems):
    gsems = sems[:NBUF]
    ssems = sems[NBUF:]
    c = lax.axis_index("c")
    s = lax.axis_index("s")
    row0 = s * ROWS_PER_TILE
    pltpu.sync_copy(zeros_hbm.at[pl.ds(row0, ROWS_PER_TILE)],
                    accum.at[pl.ds(row0, ROWS_PER_TILE)])
    pltpu.sync_copy(srcs_hbm.at[c, s], src_all)
    pltpu.sync_copy(dsts_hbm.at[s], dst_all)
    plsc.subcore_barrier()

    def gather(k, b):
        return pltpu.async_copy(table_hbm.at[src_all.at[k]], rows.at[b],
                                gsems[b])

    def scatter(k, b):
        return pltpu.async_copy(rows.at[b], accum.at[dst_all.at[k]],
                                ssems[b], add=True)

    for b in range(NBUF):
        gather(b, b)

    def group(k0, _):
        for b in range(NBUF):
            k = k0 * NBUF + b
            pltpu.make_async_copy(table_hbm.at[src_all.at[k]], rows.at[b],
                                  gsems[b]).wait()
            scatter(k, b)
            pltpu.make_async_copy(rows.at[b], accum.at[dst_all.at[k]],
                                  ssems[b]).wait()

            @pl.when(k0 < BLOCKS_PER_TILE // NBUF - 1)
            def _():
                gather(k + NBUF, b)
        return 0
    lax.fori_loop(0, BLOCKS_PER_TILE // NBUF, group, 0)

    plsc.subcore_barrier()
    pltpu.sync_copy(accum.at[pl.ds(row0, ROWS_PER_TILE)],
                    out_hbm.at[c, pl.ds(row0, ROWS_PER_TILE)])


def _sc_propagate(table_flat, srcs, dsts, zeros_rows):
    f = functools.partial(
        pl.kernel,
        out_type=jax.ShapeDtypeStruct((2, R, CQ), jnp.float32),
        mesh=_mesh(),
        scratch_types=[
            pltpu.VMEM((BLOCKS_PER_TILE, BSZ), jnp.int32),
            pltpu.VMEM((BLOCKS_PER_TILE, BSZ), jnp.int32),
            pltpu.VMEM((NBUF, BSZ, CQ), jnp.float32),
            pltpu.VMEM_SHARED((R, CQ), jnp.float32),
        ] + [pltpu.SemaphoreType.DMA] * (2 * NBUF),
        compiler_params=pltpu.CompilerParams(needs_layout_passes=False,
                                             use_tc_tiling_on_sc=False),
    )(_prop_body)
    return f(table_flat, srcs, dsts, zeros_rows)


def _mm_body(x_ref, w_ref, o_ref):
    o_ref[...] = jnp.dot(x_ref[...], w_ref[...], preferred_element_type=jnp.float32)


def _tc_matmul(x, w):
    return pl.pallas_call(
        _mm_body,
        grid=(NB,),
        in_specs=[
            pl.BlockSpec((BM, F), lambda i: (i, 0)),
            pl.BlockSpec((F, H), lambda i: (0, 0)),
        ],
        out_specs=pl.BlockSpec((BM, H), lambda i: (i, 0)),
        out_shape=jax.ShapeDtypeStruct((R, H), jnp.float32),
    )(x, w)


def _scale_body(u_ref, da_ref, db_ref, us_ref, dinv_ref):
    i = pl.program_id(0)
    rows = lax.broadcasted_iota(jnp.int32, (BM, 1), 0) + i * BM
    mask = (rows < N).astype(jnp.float32)
    dinv = lax.rsqrt(da_ref[...] + db_ref[...] + 1.0)
    dinv_ref[...] = dinv
    md = mask * dinv
    for q in range(4):
        us_ref[q] = md * u_ref[:, q * CQ:(q + 1) * CQ]


def _tc_scale(u, deg_a, deg_b):
    return pl.pallas_call(
        _scale_body,
        grid=(NB,),
        in_specs=[
            pl.BlockSpec((BM, H), lambda i: (i, 0)),
            pl.BlockSpec((BM, 1), lambda i: (i, 0)),
            pl.BlockSpec((BM, 1), lambda i: (i, 0)),
        ],
        out_specs=[
            pl.BlockSpec((4, BM, CQ), lambda i: (0, i, 0)),
            pl.BlockSpec((BM, 1), lambda i: (i, 0)),
        ],
        out_shape=[
            jax.ShapeDtypeStruct((4, R, CQ), jnp.float32),
            jax.ShapeDtypeStruct((R, 1), jnp.float32),
        ],
    )(u, deg_a, deg_b)


def _layer1_body(s1_ref, us_ref, dinv_ref, b1_ref, hs_ref):
    i = pl.program_id(0)
    rows = lax.broadcasted_iota(jnp.int32, (BM, 1), 0) + i * BM
    mask = (rows < N).astype(jnp.float32)
    dinv = dinv_ref[...]
    md = mask * dinv
    for q in range(4):
        t = dinv * (s1_ref[q] + us_ref[q]) + b1_ref[q]
        hs_ref[q] = md * jnp.tanh(t)


def _tc_layer1(s1, us3, dinv, b1_2):
    spec_q = pl.BlockSpec((4, BM, CQ), lambda i: (0, i, 0))
    return pl.pallas_call(
        _layer1_body,
        grid=(NB,),
        in_specs=[
            spec_q,
            spec_q,
            pl.BlockSpec((BM, 1), lambda i: (i, 0)),
            pl.BlockSpec((4, 1, CQ), lambda i: (0, 0, 0)),
        ],
        out_specs=spec_q,
        out_shape=jax.ShapeDtypeStruct((4, R, CQ), jnp.float32),
    )(s1, us3, dinv, b1_2)


def _head_body(s2_ref, hs_ref, dinv_ref, wm_ref, wl_ref, bm_ref, bl_ref,
               noise_ref, z_ref, mean_ref, lv_ref):
    dinv = dinv_ref[...]
    mean = bm_ref[...]
    lv = bl_ref[...]
    for q in range(4):
        pq = dinv * (s2_ref[q] + hs_ref[q])
        mean = mean + jnp.dot(pq, wm_ref[q], preferred_element_type=jnp.float32)
        lv = lv + jnp.dot(pq, wl_ref[q], preferred_element_type=jnp.float32)
    mean_ref[...] = mean
    lv_ref[...] = lv
    z_ref[...] = noise_ref[...] * jnp.exp(0.5 * lv) + mean


def _tc_head(s2, hs3, dinv, wm3, wl3, bm_2, bl_2, noise):
    spec_rl = pl.BlockSpec((4, BM, CQ), lambda i: (0, i, 0))
    return pl.pallas_call(
        _head_body,
        grid=(NB,),
        in_specs=[
            spec_rl,
            spec_rl,
            pl.BlockSpec((BM, 1), lambda i: (i, 0)),
            pl.BlockSpec((4, CQ, L), lambda i: (0, 0, 0)),
            pl.BlockSpec((4, CQ, L), lambda i: (0, 0, 0)),
            pl.BlockSpec((1, L), lambda i: (0, 0)),
            pl.BlockSpec((1, L), lambda i: (0, 0)),
            pl.BlockSpec((BM, L), lambda i: (i, 0)),
        ],
        out_specs=[
            pl.BlockSpec((BM, L), lambda i: (i, 0)),
            pl.BlockSpec((BM, L), lambda i: (i, 0)),
            pl.BlockSpec((BM, L), lambda i: (i, 0)),
        ],
        out_shape=[
            jax.ShapeDtypeStruct((R, L), jnp.float32),
            jax.ShapeDtypeStruct((R, L), jnp.float32),
            jax.ShapeDtypeStruct((R, L), jnp.float32),
        ],
    )(s2, hs3, dinv, wm3, wl3, bm_2, bl_2, noise)


@jax.jit
def _run(feature, edge_index, W1, b1, Wm, bm, Wl, bl):
    src = edge_index[0]
    dst = edge_index[1]
    pad = jnp.full((EP - E,), N, dtype=jnp.int32)
    src_p = jnp.concatenate([src, pad])
    dst_p = jnp.concatenate([dst, pad])
    srcs4 = (src_p[None, :]
             + (jnp.arange(4, dtype=jnp.int32) * R)[:, None]
             ).reshape(4, 16, BLOCKS_PER_TILE, BSZ)
    dsts_blocked = dst_p.reshape(16, BLOCKS_PER_TILE, BSZ)

    xp = jnp.zeros((R, F), jnp.float32).at[:N].set(feature)
    zeros_rows = jnp.zeros((R, CQ), jnp.float32)
    noise = jax.random.normal(jax.random.key(42), (N, L), dtype=jnp.float32)
    noise_p = jnp.zeros((R, L), jnp.float32).at[:N].set(noise)

    deg2 = _sc_degree(dst_p)
    deg_a = deg2[0].reshape(R, 1)
    deg_b = deg2[1].reshape(R, 1)

    u = _tc_matmul(xp, W1)
    us4, dinv = _tc_scale(u, deg_a, deg_b)

    ut = us4.reshape(4 * R, CQ)
    s1 = jnp.concatenate([
        _sc_propagate(ut, srcs4[:2], dsts_blocked, zeros_rows),
        _sc_propagate(ut, srcs4[2:], dsts_blocked, zeros_rows),
    ], axis=0)
    hs4 = _tc_layer1(s1, us4, dinv, b1.reshape(4, 1, CQ))
    ht = hs4.reshape(4 * R, CQ)
    s2 = jnp.concatenate([
        _sc_propagate(ht, srcs4[:2], dsts_blocked, zeros_rows),
        _sc_propagate(ht, srcs4[2:], dsts_blocked, zeros_rows),
    ], axis=0)

    z, mean, lv = _tc_head(s2, hs4, dinv, Wm.reshape(4, CQ, L), Wl.reshape(4, CQ, L),
                           bm.reshape(1, L), bl.reshape(1, L), noise_p)
    return z[:N], mean[:N], lv[:N]


def kernel(feature, edge_index, W1, b1, Wm, bm, Wl, bl):
    return _run(feature, edge_index, W1, b1, Wm, bm, Wl, bl)
